# baseline (device time: 96422 ns/iter reference)
import jax
import jax.numpy as jnp
from jax import lax
from jax.experimental import pallas as pl
from jax.experimental.pallas import tpu as pltpu

N_DEV = 32
SQ = 1024
DM = 1024
HL = 8
DH = 128
CHUNK = SQ // N_DEV
SCALE = 0.08838834764831843
BLK = 64
G = 4
RG = SQ // G
CPG = N_DEV // G


def kernel(x, Wq, K_ext, V_ext, Wo):
    def body(x_ref, wq_hbm, k_ref, v_ref, wo_hbm, out_ref,
             wq_stage, wo_stage, wqb, wob, xb, kb, vb,
             acc0, acc1, acc2, acc3, ctx_ref, rs_buf,
             w_sems, rs_send, rs_recv, ag_send, ag_recv):
        accs = (acc0, acc1, acc2, acc3)
        my_pos = lax.axis_index("i")
        col0 = my_pos * (HL * DH)

        wq_dma = pltpu.make_async_copy(
            wq_hbm.at[:, pl.ds(col0, HL * DH)], wq_stage, w_sems.at[0])
        wq_dma.start()
        wo_dma = pltpu.make_async_copy(
            wo_hbm.at[pl.ds(col0, HL * DH), :], wo_stage, w_sems.at[1])
        wo_dma.start()

        xb[...] = x_ref[...].astype(jnp.bfloat16)
        for h in range(HL):
            kb[h] = k_ref[:, h, :].astype(jnp.bfloat16)
            vb[h] = v_ref[:, h, :].astype(jnp.bfloat16)

        wq_dma.wait()
        wqb[...] = wq_stage[...].astype(jnp.bfloat16)
        wo_dma.wait()
        wob[...] = wo_stage[...].astype(jnp.bfloat16)

        rs_descs = []
        for g in range(G):
            r0 = g * RG
            kvl = (g + 1) * RG
            acc_g = accs[g]

            qg = jnp.dot(xb[r0:r0 + RG, :], wqb[...],
                         preferred_element_type=jnp.float32).astype(jnp.bfloat16)
            rb = (r0 + lax.broadcasted_iota(jnp.int32, (RG, kvl), 0)) // BLK
            cb = lax.broadcasted_iota(jnp.int32, (RG, kvl), 1) // BLK
            mask = cb <= rb
            for h in range(HL):
                qh = qg[:, h * DH:(h + 1) * DH]
                s = lax.dot_general(qh, kb[h, :kvl, :], (((1,), (1,)), ((), ())),
                                    preferred_element_type=jnp.float32) * SCALE
                w = jnp.exp(jnp.where(mask, s, -1e9))
                recip = 1.0 / jnp.sum(w, axis=1, keepdims=True)
                ctx_ref[:, h * DH:(h + 1) * DH] = (lax.dot_general(
                    w.astype(jnp.bfloat16), vb[h, :kvl, :],
                    (((1,), (0,)), ((), ())),
                    preferred_element_type=jnp.float32) * recip
                ).astype(jnp.bfloat16)
            acc_g[...] = jnp.dot(ctx_ref[...], wob[...],
                                 preferred_element_type=jnp.float32
                                 ).astype(jnp.bfloat16)

            for j in range(CPG):
                c = g * CPG + j
                d = pltpu.make_async_remote_copy(
                    src_ref=acc_g.at[pl.ds(j * CHUNK, CHUNK), :],
                    dst_ref=rs_buf.at[my_pos],
                    send_sem=rs_send.at[c],
                    recv_sem=rs_recv.at[my_pos],
                    device_id=(c,),
                    device_id_type=pl.DeviceIdType.MESH,
                )

                @pl.when(c != my_pos)
                def _(d=d):
                    d.start()

                @pl.when(c == my_pos)
                def _(acc_g=acc_g, j=j):
                    rs_buf[pl.ds(my_pos, 1), :, :] = (
                        acc_g[j * CHUNK:(j + 1) * CHUNK, :][None, :, :])

                rs_descs.append((c, d))

        for s in range(N_DEV):
            d = pltpu.make_async_remote_copy(
                src_ref=rs_buf.at[s],
                dst_ref=rs_buf.at[s],
                send_sem=rs_send.at[s],
                recv_sem=rs_recv.at[s],
                device_id=(0,),
                device_id_type=pl.DeviceIdType.MESH,
            )

            @pl.when(s != my_pos)
            def _(d=d):
                d.wait_recv()

        red = jnp.sum(rs_buf[...].astype(jnp.float32), axis=0)
        out_ref[pl.ds(my_pos * CHUNK, CHUNK), :] = red.astype(jnp.bfloat16)

        ag_descs = []
        for k in range(1, N_DEV):
            peer = lax.rem(my_pos + k, N_DEV)
            d = pltpu.make_async_remote_copy(
                src_ref=out_ref.at[pl.ds(my_pos * CHUNK, CHUNK), :],
                dst_ref=out_ref.at[pl.ds(my_pos * CHUNK, CHUNK), :],
                send_sem=ag_send.at[k],
                recv_sem=ag_recv.at[k],
                device_id=(peer,),
                device_id_type=pl.DeviceIdType.MESH,
            )
            d.start()
            ag_descs.append(d)

        for k in range(1, N_DEV):
            src = lax.rem(my_pos + (N_DEV - k), N_DEV)
            d = pltpu.make_async_remote_copy(
                src_ref=out_ref.at[pl.ds(src * CHUNK, CHUNK), :],
                dst_ref=out_ref.at[pl.ds(src * CHUNK, CHUNK), :],
                send_sem=ag_send.at[k],
                recv_sem=ag_recv.at[k],
                device_id=(my_pos,),
                device_id_type=pl.DeviceIdType.MESH,
            )
            d.wait_recv()

        for c, d in rs_descs:
            @pl.when(c != my_pos)
            def _(d=d):
                d.wait_send()
        for d in ag_descs:
            d.wait_send()

    out = pl.pallas_call(
        body,
        out_shape=jax.ShapeDtypeStruct((SQ, DM), jnp.bfloat16),
        in_specs=[
            pl.BlockSpec(memory_space=pltpu.VMEM),
            pl.BlockSpec(memory_space=pltpu.MemorySpace.HBM),
            pl.BlockSpec(memory_space=pltpu.VMEM),
            pl.BlockSpec(memory_space=pltpu.VMEM),
            pl.BlockSpec(memory_space=pltpu.MemorySpace.HBM),
        ],
        out_specs=pl.BlockSpec(memory_space=pltpu.VMEM),
        scratch_shapes=[
            pltpu.VMEM((DM, HL * DH), jnp.float32),
            pltpu.VMEM((HL * DH, DM), jnp.float32),
            pltpu.VMEM((DM, HL * DH), jnp.bfloat16),
            pltpu.VMEM((HL * DH, DM), jnp.bfloat16),
            pltpu.VMEM((SQ, DM), jnp.bfloat16),
            pltpu.VMEM((HL, SQ, DH), jnp.bfloat16),
            pltpu.VMEM((HL, SQ, DH), jnp.bfloat16),
            pltpu.VMEM((RG, DM), jnp.bfloat16),
            pltpu.VMEM((RG, DM), jnp.bfloat16),
            pltpu.VMEM((RG, DM), jnp.bfloat16),
            pltpu.VMEM((RG, DM), jnp.bfloat16),
            pltpu.VMEM((RG, HL * DH), jnp.bfloat16),
            pltpu.VMEM((N_DEV, CHUNK, DM), jnp.bfloat16),
            pltpu.SemaphoreType.DMA((2,)),
            pltpu.SemaphoreType.DMA((N_DEV,)),
            pltpu.SemaphoreType.DMA((N_DEV,)),
            pltpu.SemaphoreType.DMA((N_DEV,)),
            pltpu.SemaphoreType.DMA((N_DEV,)),
        ],
    )(x[0], Wq, K_ext[0], V_ext[0], Wo)
    return out[None]
